# Initial kernel scaffold; baseline (speedup 1.0000x reference)
import jax
import jax.numpy as jnp
from jax.experimental import pallas as pl


def kernel(inputs, indexes, positions):
    raise NotImplementedError("write your pallas kernel here")


# speedup vs baseline: 10.7061x; 10.7061x over previous
import functools

import jax
import jax.numpy as jnp
from jax import lax
from jax.experimental import pallas as pl
from jax.experimental.pallas import tpu as pltpu
from jax.experimental.pallas import tpu_sc as plsc

N_CELLS = 100000
STENCIL = 25
BATCH = 16

NC = 2
NS = 16
L = 16
NW = NC * NS

FLAT = N_CELLS * STENCIL
TILE_RANGE = (FLAT + NW - 1) // NW
TILE_RANGE = ((TILE_RANGE + L - 1) // L) * L
FLAT_PAD = NW * TILE_RANGE
SENTINEL = N_CELLS
TBL_W = N_CELLS + L
PCHUNK = 2048
HALF = FLAT // 2
K = 2000
NCHUNK = HALF // K


def _mesh():
    return plsc.VectorSubcoreMesh(
        core_axis_name="c", subcore_axis_name="s", num_cores=NC, num_subcores=NS
    )


def _remap_body(n_pchunks, idxf_hbm, pf_hbm, idx2_hbm, buf, pbuf):
    wid = lax.axis_index("s") * NC + lax.axis_index("c")
    base = wid * TILE_RANGE
    pltpu.sync_copy(idxf_hbm.at[pl.ds(base, TILE_RANGE)], buf)
    sent = jnp.full((L,), SENTINEL, jnp.int32)

    def pchunk(j, carry):
        pltpu.sync_copy(pf_hbm.at[pl.ds(j * PCHUNK, PCHUNK)], pbuf)

        def inner(i, c2):
            v = pbuf[pl.ds(i * L, L)]
            msk = (v >= base) & (v < base + TILE_RANGE)
            local = jnp.where(msk, v - base, 0)
            plsc.store_scatter(buf, [local], sent, mask=msk)
            return c2

        return lax.fori_loop(0, PCHUNK // L, inner, carry)

    lax.fori_loop(0, n_pchunks, pchunk, 0)
    pltpu.sync_copy(buf, idx2_hbm.at[pl.ds(base, TILE_RANGE)])


def _gather_body(tbl_hbm, idx2_hbm, out_hbm, table, ibuf, obuf):
    wid = lax.axis_index("s") * NC + lax.axis_index("c")
    b = wid // 2
    h = wid % 2
    pltpu.sync_copy(tbl_hbm.at[pl.ds(b * TBL_W, TBL_W)], table)
    hbase = h * HALF

    def chunk(c, carry):
        off = hbase + c * K
        pltpu.sync_copy(idx2_hbm.at[pl.ds(off, K)], ibuf)

        def inner(i, c2):
            v = ibuf[pl.ds(i * L, L)]
            obuf[pl.ds(i * L, L)] = plsc.load_gather(table, [v])
            return c2

        lax.fori_loop(0, K // L, inner, 0)
        pltpu.sync_copy(obuf, out_hbm.at[pl.ds(b * FLAT + off, K)])
        return carry

    lax.fori_loop(0, NCHUNK, chunk, 0)


def kernel(inputs, indexes, positions):
    pf = positions[:, 0] * STENCIL + positions[:, 1]
    p = pf.shape[0]
    n_pchunks = max(1, (p + PCHUNK - 1) // PCHUNK)
    pf_pad = jnp.concatenate(
        [pf.astype(jnp.int32), jnp.full((n_pchunks * PCHUNK - p,), -1, jnp.int32)]
    )
    idxf = jnp.concatenate(
        [indexes.reshape(-1), jnp.zeros((FLAT_PAD - FLAT,), jnp.int32)]
    )
    tbl = jnp.concatenate(
        [inputs, jnp.zeros((BATCH, L), jnp.float32)], axis=1
    ).reshape(-1)

    idx2 = pl.kernel(
        functools.partial(_remap_body, n_pchunks),
        out_type=jax.ShapeDtypeStruct((FLAT_PAD,), jnp.int32),
        mesh=_mesh(),
        scratch_types=[
            pltpu.VMEM((TILE_RANGE,), jnp.int32),
            pltpu.VMEM((PCHUNK,), jnp.int32),
        ],
        compiler_params=pltpu.CompilerParams(needs_layout_passes=False),
    )(idxf, pf_pad)

    out = pl.kernel(
        _gather_body,
        out_type=jax.ShapeDtypeStruct((BATCH * FLAT,), jnp.float32),
        mesh=_mesh(),
        scratch_types=[
            pltpu.VMEM((TBL_W,), jnp.float32),
            pltpu.VMEM((K,), jnp.int32),
            pltpu.VMEM((K,), jnp.float32),
        ],
        compiler_params=pltpu.CompilerParams(needs_layout_passes=False),
    )(tbl, idx2)

    return out.reshape(BATCH, N_CELLS, STENCIL)


# speedup vs baseline: 14.1879x; 1.3252x over previous
import functools

import jax
import jax.numpy as jnp
from jax import lax
from jax.experimental import pallas as pl
from jax.experimental.pallas import tpu as pltpu
from jax.experimental.pallas import tpu_sc as plsc

N_CELLS = 100000
STENCIL = 25
BATCH = 16

NC = 2
NS = 16
L = 16
NW = NC * NS

FLAT = N_CELLS * STENCIL
TILE_RANGE = (FLAT + NW - 1) // NW
TILE_RANGE = ((TILE_RANGE + L - 1) // L) * L
SENTINEL = N_CELLS
TBL_W = N_CELLS + L
PCHUNK = 2048
HALF = FLAT // 2
K = 2000
NCHUNK = HALF // K
NPAIR = (NCHUNK - 1) // 2
UNROLL = 5


def _mesh():
    return plsc.VectorSubcoreMesh(
        core_axis_name="c", subcore_axis_name="s", num_cores=NC, num_subcores=NS
    )


def _remap_body(n_pchunks, idxf_hbm, pf_hbm, idx2_hbm, buf, pbuf):
    wid = lax.axis_index("s") * NC + lax.axis_index("c")
    base = jnp.minimum(wid * TILE_RANGE, FLAT - TILE_RANGE)
    pltpu.sync_copy(idxf_hbm.at[pl.ds(base, TILE_RANGE)], buf)
    sent = jnp.full((L,), SENTINEL, jnp.int32)

    def pchunk(j, carry):
        pltpu.sync_copy(pf_hbm.at[pl.ds(j * PCHUNK, PCHUNK)], pbuf)

        def inner(i, c2):
            for u in range(4):
                v = pbuf[pl.ds((i * 4 + u) * L, L)]
                msk = (v >= base) & (v < base + TILE_RANGE)
                local = jnp.where(msk, v - base, 0)
                plsc.store_scatter(buf, [local], sent, mask=msk)
            return c2

        return lax.fori_loop(0, PCHUNK // L // 4, inner, carry)

    lax.fori_loop(0, n_pchunks, pchunk, 0)
    pltpu.sync_copy(buf, idx2_hbm.at[pl.ds(base, TILE_RANGE)])


def _gather_chunk(table, ibuf, obuf):
    def inner(i, c2):
        for u in range(UNROLL):
            o = (i * UNROLL + u) * L
            v = ibuf[pl.ds(o, L)]
            obuf[pl.ds(o, L)] = plsc.load_gather(table, [v])
        return c2

    lax.fori_loop(0, K // L // UNROLL, inner, 0)


def _gather_body(
    inp_hbm, idx2_hbm, out_hbm, table, ib0, ib1, ob0, ob1, is0, is1, os0, os1
):
    wid = lax.axis_index("s") * NC + lax.axis_index("c")
    b = wid // 2
    h = wid % 2
    hbase = h * HALF
    obase = b * FLAT + hbase

    def start_in(c, ib, sem):
        pltpu.async_copy(idx2_hbm.at[pl.ds(hbase + c * K, K)], ib, sem)

    def wait_in(c, ib, sem):
        pltpu.make_async_copy(idx2_hbm.at[pl.ds(hbase + c * K, K)], ib, sem).wait()

    def start_out(c, ob, sem):
        pltpu.async_copy(ob, out_hbm.at[pl.ds(obase + c * K, K)], sem)

    def wait_out(c, ob, sem):
        pltpu.make_async_copy(ob, out_hbm.at[pl.ds(obase + c * K, K)], sem).wait()

    start_in(0, ib0, is0)
    start_in(1, ib1, is1)
    pltpu.sync_copy(
        inp_hbm.at[pl.ds(b * N_CELLS, N_CELLS)], table.at[pl.ds(0, N_CELLS)]
    )
    table[pl.ds(N_CELLS, L)] = jnp.zeros((L,), jnp.float32)

    def pair(j, carry):
        a = 2 * j
        wait_in(a, ib0, is0)

        @pl.when(j > 0)
        def _():
            wait_out(a - 2, ob0, os0)

        _gather_chunk(table, ib0, ob0)
        start_out(a, ob0, os0)
        start_in(a + 2, ib0, is0)

        wait_in(a + 1, ib1, is1)

        @pl.when(j > 0)
        def _():
            wait_out(a - 1, ob1, os1)

        _gather_chunk(table, ib1, ob1)
        start_out(a + 1, ob1, os1)

        @pl.when(j < NPAIR - 1)
        def _():
            start_in(a + 3, ib1, is1)

        return carry

    lax.fori_loop(0, NPAIR, pair, 0)

    wait_in(NCHUNK - 1, ib0, is0)
    wait_out(NCHUNK - 3, ob0, os0)
    _gather_chunk(table, ib0, ob0)
    start_out(NCHUNK - 1, ob0, os0)
    wait_out(NCHUNK - 2, ob1, os1)
    wait_out(NCHUNK - 1, ob0, os0)


def kernel(inputs, indexes, positions):
    pf = positions[:, 0] * STENCIL + positions[:, 1]
    p = pf.shape[0]
    n_pchunks = max(1, (p + PCHUNK - 1) // PCHUNK)
    pf_pad = jnp.concatenate(
        [pf.astype(jnp.int32), jnp.full((n_pchunks * PCHUNK - p,), -1, jnp.int32)]
    )
    idxf = indexes.reshape(-1)
    inp = inputs.reshape(-1)

    idx2 = pl.kernel(
        functools.partial(_remap_body, n_pchunks),
        out_type=jax.ShapeDtypeStruct((FLAT,), jnp.int32),
        mesh=_mesh(),
        scratch_types=[
            pltpu.VMEM((TILE_RANGE,), jnp.int32),
            pltpu.VMEM((PCHUNK,), jnp.int32),
        ],
        compiler_params=pltpu.CompilerParams(needs_layout_passes=False),
    )(idxf, pf_pad)

    out = pl.kernel(
        _gather_body,
        out_type=jax.ShapeDtypeStruct((BATCH * FLAT,), jnp.float32),
        mesh=_mesh(),
        scratch_types=[
            pltpu.VMEM((TBL_W,), jnp.float32),
            pltpu.VMEM((K,), jnp.int32),
            pltpu.VMEM((K,), jnp.int32),
            pltpu.VMEM((K,), jnp.float32),
            pltpu.VMEM((K,), jnp.float32),
            pltpu.SemaphoreType.DMA,
            pltpu.SemaphoreType.DMA,
            pltpu.SemaphoreType.DMA,
            pltpu.SemaphoreType.DMA,
        ],
        compiler_params=pltpu.CompilerParams(needs_layout_passes=False),
    )(inp, idx2)

    return out.reshape(BATCH, N_CELLS, STENCIL)


# speedup vs baseline: 18.3646x; 1.2944x over previous
import functools

import jax
import jax.numpy as jnp
from jax import lax
from jax.experimental import pallas as pl
from jax.experimental.pallas import tpu as pltpu
from jax.experimental.pallas import tpu_sc as plsc

N_CELLS = 100000
STENCIL = 25
BATCH = 16

NC = 2
NS = 16
L = 16
NW = NC * NS

FLAT = N_CELLS * STENCIL
TILE_RANGE = (FLAT + NW - 1) // NW
TILE_RANGE = ((TILE_RANGE + L - 1) // L) * L
SENTINEL = N_CELLS
TBL_W = N_CELLS + L
PCHUNK = 2048
HALF = FLAT // 2
K = 2000
NCHUNK = HALF // K
NPAIR = (NCHUNK - 1) // 2
UNROLL = 5


def _mesh():
    return plsc.VectorSubcoreMesh(
        core_axis_name="c", subcore_axis_name="s", num_cores=NC, num_subcores=NS
    )


def _remap_body(n_pchunks, idxf_hbm, pf_hbm, idx2_hbm, buf, pbuf):
    wid = lax.axis_index("s") * NC + lax.axis_index("c")
    base = jnp.minimum(wid * TILE_RANGE, FLAT - TILE_RANGE)
    pltpu.sync_copy(idxf_hbm.at[pl.ds(base, TILE_RANGE)], buf)
    sent = jnp.full((L,), SENTINEL, jnp.int32)

    def pchunk(j, carry):
        pltpu.sync_copy(pf_hbm.at[pl.ds(j * PCHUNK, PCHUNK)], pbuf)

        def inner(i, c2):
            for u in range(4):
                v = pbuf[pl.ds((i * 4 + u) * L, L)]
                msk = (v >= base) & (v < base + TILE_RANGE)
                local = jnp.where(msk, v - base, 0)
                plsc.store_scatter(buf, [local], sent, mask=msk)
            return c2

        return lax.fori_loop(0, PCHUNK // L // 4, inner, carry)

    lax.fori_loop(0, n_pchunks, pchunk, 0)
    pltpu.sync_copy(buf, idx2_hbm.at[pl.ds(base, TILE_RANGE)])


def _gather_chunk(table, ibuf, obuf):
    lanes = lax.iota(jnp.int32, L)

    def inner(i, c2):
        for u in range(UNROLL):
            o = (i * UNROLL + u) * L
            v = ibuf[pl.ds(o, L)]
            flat = o + lanes
            rows = flat // STENCIL
            cols = flat - rows * STENCIL
            plsc.store_scatter(obuf, [rows, cols], plsc.load_gather(table, [v]))
        return c2

    lax.fori_loop(0, K // L // UNROLL, inner, 0)


def _gather_body(
    inp_hbm, idx2_hbm, out_hbm, table, ib0, ib1, ob0, ob1, is0, is1, os0, os1
):
    wid = lax.axis_index("s") * NC + lax.axis_index("c")
    b = wid // 2
    h = wid % 2
    hbase = h * HALF
    CC = K // STENCIL
    cbase = h * (N_CELLS // 2)

    def start_in(c, ib, sem):
        pltpu.async_copy(idx2_hbm.at[pl.ds(hbase + c * K, K)], ib, sem)

    def wait_in(c, ib, sem):
        pltpu.make_async_copy(idx2_hbm.at[pl.ds(hbase + c * K, K)], ib, sem).wait()

    def start_out(c, ob, sem):
        pltpu.async_copy(ob, out_hbm.at[b, pl.ds(cbase + c * CC, CC), :], sem)

    def wait_out(c, ob, sem):
        pltpu.make_async_copy(
            ob, out_hbm.at[b, pl.ds(cbase + c * CC, CC), :], sem
        ).wait()

    start_in(0, ib0, is0)
    start_in(1, ib1, is1)
    pltpu.sync_copy(
        inp_hbm.at[pl.ds(b * N_CELLS, N_CELLS)], table.at[pl.ds(0, N_CELLS)]
    )
    table[pl.ds(N_CELLS, L)] = jnp.zeros((L,), jnp.float32)

    def pair(j, carry):
        a = 2 * j
        wait_in(a, ib0, is0)

        @pl.when(j > 0)
        def _():
            wait_out(a - 2, ob0, os0)

        _gather_chunk(table, ib0, ob0)
        start_out(a, ob0, os0)
        start_in(a + 2, ib0, is0)

        wait_in(a + 1, ib1, is1)

        @pl.when(j > 0)
        def _():
            wait_out(a - 1, ob1, os1)

        _gather_chunk(table, ib1, ob1)
        start_out(a + 1, ob1, os1)

        @pl.when(j < NPAIR - 1)
        def _():
            start_in(a + 3, ib1, is1)

        return carry

    lax.fori_loop(0, NPAIR, pair, 0)

    wait_in(NCHUNK - 1, ib0, is0)
    wait_out(NCHUNK - 3, ob0, os0)
    _gather_chunk(table, ib0, ob0)
    start_out(NCHUNK - 1, ob0, os0)
    wait_out(NCHUNK - 2, ob1, os1)
    wait_out(NCHUNK - 1, ob0, os0)


def kernel(inputs, indexes, positions):
    pf = positions[:, 0] * STENCIL + positions[:, 1]
    p = pf.shape[0]
    n_pchunks = max(1, (p + PCHUNK - 1) // PCHUNK)
    pf_pad = jnp.concatenate(
        [pf.astype(jnp.int32), jnp.full((n_pchunks * PCHUNK - p,), -1, jnp.int32)]
    )
    idxf = indexes.reshape(-1)
    inp = inputs.reshape(-1)

    idx2 = pl.kernel(
        functools.partial(_remap_body, n_pchunks),
        out_type=jax.ShapeDtypeStruct((FLAT,), jnp.int32),
        mesh=_mesh(),
        scratch_types=[
            pltpu.VMEM((TILE_RANGE,), jnp.int32),
            pltpu.VMEM((PCHUNK,), jnp.int32),
        ],
        compiler_params=pltpu.CompilerParams(needs_layout_passes=False),
    )(idxf, pf_pad)

    out = pl.kernel(
        _gather_body,
        out_type=jax.ShapeDtypeStruct((BATCH, N_CELLS, STENCIL), jnp.float32),
        mesh=_mesh(),
        scratch_types=[
            pltpu.VMEM((TBL_W,), jnp.float32),
            pltpu.VMEM((K,), jnp.int32),
            pltpu.VMEM((K,), jnp.int32),
            pltpu.VMEM((K // STENCIL, STENCIL), jnp.float32),
            pltpu.VMEM((K // STENCIL, STENCIL), jnp.float32),
            pltpu.SemaphoreType.DMA,
            pltpu.SemaphoreType.DMA,
            pltpu.SemaphoreType.DMA,
            pltpu.SemaphoreType.DMA,
        ],
        compiler_params=pltpu.CompilerParams(needs_layout_passes=False),
    )(inp, idx2)

    return out


# speedup vs baseline: 18.8561x; 1.0268x over previous
import functools

import jax
import jax.numpy as jnp
from jax import lax
from jax.experimental import pallas as pl
from jax.experimental.pallas import tpu as pltpu
from jax.experimental.pallas import tpu_sc as plsc

N_CELLS = 100000
STENCIL = 25
BATCH = 16

NC = 2
NS = 16
L = 16
NW = NC * NS

FLAT = N_CELLS * STENCIL
TILE_RANGE = (FLAT + NW - 1) // NW
TILE_RANGE = ((TILE_RANGE + L - 1) // L) * L
SENTINEL = N_CELLS
TBL_W = N_CELLS + L
PCHUNK = 2048
CC = 128
CW = CC * STENCIL
NQ = N_CELLS // CC
TAILC = N_CELLS - NQ * CC
TAILW = TAILC * STENCIL
NPAIR = 195
UNROLL = 5


def _mesh():
    return plsc.VectorSubcoreMesh(
        core_axis_name="c", subcore_axis_name="s", num_cores=NC, num_subcores=NS
    )


def _remap_body(n_pchunks, idxf_hbm, pf_hbm, idx2_hbm, buf, pbuf):
    wid = lax.axis_index("s") * NC + lax.axis_index("c")
    base = jnp.minimum(wid * TILE_RANGE, FLAT - TILE_RANGE)
    pltpu.sync_copy(idxf_hbm.at[pl.ds(base, TILE_RANGE)], buf)
    sent = jnp.full((L,), SENTINEL, jnp.int32)

    def pchunk(j, carry):
        pltpu.sync_copy(pf_hbm.at[pl.ds(j * PCHUNK, PCHUNK)], pbuf)

        def inner(i, c2):
            for u in range(4):
                v = pbuf[pl.ds((i * 4 + u) * L, L)]
                msk = (v >= base) & (v < base + TILE_RANGE)
                local = jnp.where(msk, v - base, 0)
                plsc.store_scatter(buf, [local], sent, mask=msk)
            return c2

        return lax.fori_loop(0, PCHUNK // L // 4, inner, carry)

    lax.fori_loop(0, n_pchunks, pchunk, 0)
    pltpu.sync_copy(buf, idx2_hbm.at[pl.ds(base, TILE_RANGE)])


def _gather_chunk(table, ibuf, obuf, n_windows):
    lanes = lax.iota(jnp.int32, L)

    def inner(i, c2):
        for u in range(UNROLL):
            o = (i * UNROLL + u) * L
            v = ibuf[pl.ds(o, L)]
            flat = o + lanes
            cols = flat // STENCIL
            rows = flat - cols * STENCIL
            plsc.store_scatter(obuf, [rows, cols], plsc.load_gather(table, [v]))
        return c2

    lax.fori_loop(0, n_windows // UNROLL, inner, 0)


def _gather_body(
    inp_hbm,
    idx2_hbm,
    out_hbm,
    table,
    ib0,
    ib1,
    ibt,
    ob0,
    ob1,
    obt,
    is0,
    is1,
    ist,
    os0,
    os1,
    ost,
):
    wid = lax.axis_index("s") * NC + lax.axis_index("c")
    b = wid // 2
    h = wid % 2

    def start_in(q, ib, sem):
        pltpu.async_copy(idx2_hbm.at[pl.ds(q * CW, CW)], ib, sem)

    def wait_in(q, ib, sem):
        pltpu.make_async_copy(idx2_hbm.at[pl.ds(q * CW, CW)], ib, sem).wait()

    def start_out(q, ob, sem):
        pltpu.async_copy(ob, out_hbm.at[b, :, pl.ds(q * CC, CC)], sem)

    def wait_out(q, ob, sem):
        pltpu.make_async_copy(ob, out_hbm.at[b, :, pl.ds(q * CC, CC)], sem).wait()

    start_in(h, ib0, is0)
    start_in(h + 2, ib1, is1)
    pltpu.sync_copy(
        inp_hbm.at[pl.ds(b * N_CELLS, N_CELLS)], table.at[pl.ds(0, N_CELLS)]
    )
    table[pl.ds(N_CELLS, L)] = jnp.zeros((L,), jnp.float32)

    @pl.when(h == 1)
    def _():
        pltpu.sync_copy(idx2_hbm.at[pl.ds(NQ * CW, TAILW)], ibt)
        _gather_chunk(table, ibt, obt, TAILW // L)
        pltpu.async_copy(obt, out_hbm.at[b, :, pl.ds(NQ * CC, TAILC)], ost)

    def pair(j, carry):
        q0 = 4 * j + h
        wait_in(q0, ib0, is0)

        @pl.when(j > 0)
        def _():
            wait_out(q0 - 4, ob0, os0)

        _gather_chunk(table, ib0, ob0, CW // L)
        start_out(q0, ob0, os0)

        @pl.when(q0 + 4 < NQ)
        def _():
            start_in(q0 + 4, ib0, is0)

        q1 = q0 + 2
        wait_in(q1, ib1, is1)

        @pl.when(j > 0)
        def _():
            wait_out(q1 - 4, ob1, os1)

        _gather_chunk(table, ib1, ob1, CW // L)
        start_out(q1, ob1, os1)

        @pl.when(q1 + 4 < NQ)
        def _():
            start_in(q1 + 4, ib1, is1)

        return carry

    lax.fori_loop(0, NPAIR, pair, 0)

    @pl.when(h == 0)
    def _():
        wait_in(NQ - 1, ib0, is0)
        wait_out(NQ - 5, ob0, os0)
        _gather_chunk(table, ib0, ob0, CW // L)
        start_out(NQ - 1, ob0, os0)

    wait_out(jnp.where(h == 0, NQ - 1, NQ - 4), ob0, os0)
    wait_out(NQ - 3 + h, ob1, os1)

    @pl.when(h == 1)
    def _():
        pltpu.make_async_copy(
            obt, out_hbm.at[b, :, pl.ds(NQ * CC, TAILC)], ost
        ).wait()


def kernel(inputs, indexes, positions):
    pf = positions[:, 0] * STENCIL + positions[:, 1]
    p = pf.shape[0]
    n_pchunks = max(1, (p + PCHUNK - 1) // PCHUNK)
    pf_pad = jnp.concatenate(
        [pf.astype(jnp.int32), jnp.full((n_pchunks * PCHUNK - p,), -1, jnp.int32)]
    )
    idxf = indexes.reshape(-1)
    inp = inputs.reshape(-1)

    idx2 = pl.kernel(
        functools.partial(_remap_body, n_pchunks),
        out_type=jax.ShapeDtypeStruct((FLAT,), jnp.int32),
        mesh=_mesh(),
        scratch_types=[
            pltpu.VMEM((TILE_RANGE,), jnp.int32),
            pltpu.VMEM((PCHUNK,), jnp.int32),
        ],
        compiler_params=pltpu.CompilerParams(needs_layout_passes=False),
    )(idxf, pf_pad)

    out = pl.kernel(
        _gather_body,
        out_type=jax.ShapeDtypeStruct((BATCH, STENCIL, N_CELLS), jnp.float32),
        mesh=_mesh(),
        scratch_types=[
            pltpu.VMEM((TBL_W,), jnp.float32),
            pltpu.VMEM((CW,), jnp.int32),
            pltpu.VMEM((CW,), jnp.int32),
            pltpu.VMEM((TAILW,), jnp.int32),
            pltpu.VMEM((STENCIL, CC), jnp.float32),
            pltpu.VMEM((STENCIL, CC), jnp.float32),
            pltpu.VMEM((STENCIL, TAILC), jnp.float32),
            pltpu.SemaphoreType.DMA,
            pltpu.SemaphoreType.DMA,
            pltpu.SemaphoreType.DMA,
            pltpu.SemaphoreType.DMA,
            pltpu.SemaphoreType.DMA,
            pltpu.SemaphoreType.DMA,
        ],
        compiler_params=pltpu.CompilerParams(needs_layout_passes=False),
    )(inp, idx2)

    return out.transpose(0, 2, 1)


# speedup vs baseline: 41.5791x; 2.2051x over previous
import functools

import jax
import jax.numpy as jnp
from jax import lax
from jax.experimental import pallas as pl
from jax.experimental.pallas import tpu as pltpu
from jax.experimental.pallas import tpu_sc as plsc

N_CELLS = 100000
STENCIL = 25
BATCH = 16

NC = 2
NS = 16
L = 16
NW = NC * NS

SENTINEL = N_CELLS
TBL_W = N_CELLS + L
PCHUNK = 2048
CC = 128
CW = CC * STENCIL
NQ = N_CELLS // CC
TAILC = N_CELLS - NQ * CC
KB = 25
TAIL_OWNER = NQ - (NQ // NW) * NW
NPAIR = 195


def _mesh():
    return plsc.VectorSubcoreMesh(
        core_axis_name="c", subcore_axis_name="s", num_cores=NC, num_subcores=NS
    )


def _remap_body(
    n_pchunks, idxT_hbm, pr_hbm, ps_hbm, idx2_hbm, buf3, buft, buft32, prb, psb
):
    wid = lax.axis_index("s") * NC + lax.axis_index("c")
    sent = jnp.full((L,), SENTINEL, jnp.int32)

    def ldblk(k, carry):
        q = wid + NW * k

        @pl.when(q < NQ)
        def _():
            pltpu.sync_copy(
                idxT_hbm.at[:, pl.ds(q * CC, CC)],
                buf3.at[pl.ds(k * STENCIL, STENCIL)],
            )

        @pl.when(q == NQ)
        def _():
            pltpu.sync_copy(idxT_hbm.at[:, pl.ds(NQ * CC, TAILC)], buft32)
            for s in range(STENCIL):
                for w in range(CC // L):
                    if w < TAILC // L:
                        buft[s, pl.ds(w * L, L)] = buft32[s, pl.ds(w * L, L)]
                    else:
                        buft[s, pl.ds(w * L, L)] = sent

        return carry

    lax.fori_loop(0, KB, ldblk, 0)

    def pchunk(j, carry):
        pltpu.sync_copy(pr_hbm.at[pl.ds(j * PCHUNK, PCHUNK)], prb)
        pltpu.sync_copy(ps_hbm.at[pl.ds(j * PCHUNK, PCHUNK)], psb)

        def inner(i, c2):
            for u in range(4):
                o = (i * 4 + u) * L
                r = prb[pl.ds(o, L)]
                s = psb[pl.ds(o, L)]
                q = r >> 7
                k = q >> 5
                c = r & (CC - 1)
                mfull = (r >= 0) & (q < NQ) & ((q & (NW - 1)) == wid)
                plsc.store_scatter(
                    buf3,
                    [jnp.where(mfull, k * STENCIL + s, 0), jnp.where(mfull, c, 0)],
                    sent,
                    mask=mfull,
                )
                mtail = (r >= 0) & (q == NQ) & (wid == TAIL_OWNER)
                plsc.store_scatter(
                    buft,
                    [s, jnp.where(mtail, c, 0)],
                    sent,
                    mask=mtail,
                )
            return c2

        return lax.fori_loop(0, PCHUNK // L // 4, inner, carry)

    lax.fori_loop(0, n_pchunks, pchunk, 0)

    def wrblk(k, carry):
        q = wid + NW * k

        @pl.when(q < NQ)
        def _():
            pltpu.sync_copy(buf3.at[pl.ds(k * STENCIL, STENCIL)], idx2_hbm.at[q])

        @pl.when(q == NQ)
        def _():
            pltpu.sync_copy(buft, idx2_hbm.at[NQ])

        return carry

    lax.fori_loop(0, KB, wrblk, 0)


def _gather_chunk(table, ibuf, obuf, width):
    for s in range(STENCIL):
        for w in range(width // L):
            v = ibuf[s, pl.ds(w * L, L)]
            obuf[s, pl.ds(w * L, L)] = plsc.load_gather(table, [v])


def _gather_body(
    inp_hbm,
    idx2_hbm,
    out_hbm,
    table,
    ib0,
    ib1,
    ibt,
    ob0,
    ob1,
    obt,
    is0,
    is1,
    ist,
    os0,
    os1,
    ost,
):
    wid = lax.axis_index("s") * NC + lax.axis_index("c")
    b = wid // 2
    h = wid % 2

    def start_in(q, ib, sem):
        pltpu.async_copy(idx2_hbm.at[q], ib, sem)

    def wait_in(q, ib, sem):
        pltpu.make_async_copy(idx2_hbm.at[q], ib, sem).wait()

    def start_out(q, ob, sem):
        pltpu.async_copy(ob, out_hbm.at[b, :, pl.ds(q * CC, CC)], sem)

    def wait_out(q, ob, sem):
        pltpu.make_async_copy(ob, out_hbm.at[b, :, pl.ds(q * CC, CC)], sem).wait()

    start_in(h, ib0, is0)
    start_in(h + 2, ib1, is1)
    pltpu.sync_copy(
        inp_hbm.at[pl.ds(b * N_CELLS, N_CELLS)], table.at[pl.ds(0, N_CELLS)]
    )
    table[pl.ds(N_CELLS, L)] = jnp.zeros((L,), jnp.float32)

    @pl.when(h == 1)
    def _():
        pltpu.sync_copy(idx2_hbm.at[NQ], ibt)
        _gather_chunk(table, ibt, obt, TAILC)
        pltpu.async_copy(obt, out_hbm.at[b, :, pl.ds(NQ * CC, TAILC)], ost)

    def pair(j, carry):
        q0 = 4 * j + h
        wait_in(q0, ib0, is0)

        @pl.when(j > 0)
        def _():
            wait_out(q0 - 4, ob0, os0)

        _gather_chunk(table, ib0, ob0, CC)
        start_out(q0, ob0, os0)

        @pl.when(q0 + 4 < NQ)
        def _():
            start_in(q0 + 4, ib0, is0)

        q1 = q0 + 2
        wait_in(q1, ib1, is1)

        @pl.when(j > 0)
        def _():
            wait_out(q1 - 4, ob1, os1)

        _gather_chunk(table, ib1, ob1, CC)
        start_out(q1, ob1, os1)

        @pl.when(q1 + 4 < NQ)
        def _():
            start_in(q1 + 4, ib1, is1)

        return carry

    lax.fori_loop(0, NPAIR, pair, 0)

    @pl.when(h == 0)
    def _():
        wait_in(NQ - 1, ib0, is0)
        wait_out(NQ - 5, ob0, os0)
        _gather_chunk(table, ib0, ob0, CC)
        start_out(NQ - 1, ob0, os0)

    wait_out(jnp.where(h == 0, NQ - 1, NQ - 4), ob0, os0)
    wait_out(NQ - 3 + h, ob1, os1)

    @pl.when(h == 1)
    def _():
        pltpu.make_async_copy(
            obt, out_hbm.at[b, :, pl.ds(NQ * CC, TAILC)], ost
        ).wait()


def kernel(inputs, indexes, positions):
    idxT = indexes.T
    pr = positions[:, 0].astype(jnp.int32)
    ps = positions[:, 1].astype(jnp.int32)
    p = pr.shape[0]
    n_pchunks = max(1, (p + PCHUNK - 1) // PCHUNK)
    pad = jnp.full((n_pchunks * PCHUNK - p,), -1, jnp.int32)
    pr_pad = jnp.concatenate([pr, pad])
    ps_pad = jnp.concatenate([ps, pad])
    inp = inputs.reshape(-1)

    idx2 = pl.kernel(
        functools.partial(_remap_body, n_pchunks),
        out_type=jax.ShapeDtypeStruct((NQ + 1, STENCIL, CC), jnp.int32),
        mesh=_mesh(),
        scratch_types=[
            pltpu.VMEM((KB * STENCIL, CC), jnp.int32),
            pltpu.VMEM((STENCIL, CC), jnp.int32),
            pltpu.VMEM((STENCIL, TAILC), jnp.int32),
            pltpu.VMEM((PCHUNK,), jnp.int32),
            pltpu.VMEM((PCHUNK,), jnp.int32),
        ],
        compiler_params=pltpu.CompilerParams(needs_layout_passes=False),
    )(idxT, pr_pad, ps_pad)

    out = pl.kernel(
        _gather_body,
        out_type=jax.ShapeDtypeStruct((BATCH, STENCIL, N_CELLS), jnp.float32),
        mesh=_mesh(),
        scratch_types=[
            pltpu.VMEM((TBL_W,), jnp.float32),
            pltpu.VMEM((STENCIL, CC), jnp.int32),
            pltpu.VMEM((STENCIL, CC), jnp.int32),
            pltpu.VMEM((STENCIL, CC), jnp.int32),
            pltpu.VMEM((STENCIL, CC), jnp.float32),
            pltpu.VMEM((STENCIL, CC), jnp.float32),
            pltpu.VMEM((STENCIL, TAILC), jnp.float32),
            pltpu.SemaphoreType.DMA,
            pltpu.SemaphoreType.DMA,
            pltpu.SemaphoreType.DMA,
            pltpu.SemaphoreType.DMA,
            pltpu.SemaphoreType.DMA,
            pltpu.SemaphoreType.DMA,
        ],
        compiler_params=pltpu.CompilerParams(needs_layout_passes=False),
    )(inp, idx2)

    return out.transpose(0, 2, 1)


# speedup vs baseline: 41.9566x; 1.0091x over previous
import functools

import jax
import jax.numpy as jnp
from jax import lax
from jax.experimental import pallas as pl
from jax.experimental.pallas import tpu as pltpu
from jax.experimental.pallas import tpu_sc as plsc

N_CELLS = 100000
STENCIL = 25
BATCH = 16

NC = 2
NS = 16
L = 16
NW = NC * NS

SENTINEL = N_CELLS
TBL_W = N_CELLS + L
PCHUNK = 2048
CC = 128
CW = CC * STENCIL
NQ = N_CELLS // CC
TAILC = N_CELLS - NQ * CC
KB = 25
TAIL_OWNER = NQ - (NQ // NW) * NW
NPAIR = 195


def _mesh():
    return plsc.VectorSubcoreMesh(
        core_axis_name="c", subcore_axis_name="s", num_cores=NC, num_subcores=NS
    )


def _remap_body(
    n_pchunks, idxT_hbm, pr_hbm, ps_hbm, idx2_hbm, buf3, buft, buft32, prb, psb
):
    wid = lax.axis_index("s") * NC + lax.axis_index("c")
    sent = jnp.full((L,), SENTINEL, jnp.int32)

    def ldblk(k, carry):
        q = wid + NW * k

        @pl.when(q < NQ)
        def _():
            pltpu.sync_copy(
                idxT_hbm.at[:, pl.ds(q * CC, CC)],
                buf3.at[pl.ds(k * STENCIL, STENCIL)],
            )

        @pl.when(q == NQ)
        def _():
            pltpu.sync_copy(idxT_hbm.at[:, pl.ds(NQ * CC, TAILC)], buft32)
            for s in range(STENCIL):
                for w in range(CC // L):
                    if w < TAILC // L:
                        buft[s, pl.ds(w * L, L)] = buft32[s, pl.ds(w * L, L)]
                    else:
                        buft[s, pl.ds(w * L, L)] = sent

        return carry

    lax.fori_loop(0, KB, ldblk, 0)

    def pchunk(j, carry):
        pltpu.sync_copy(pr_hbm.at[pl.ds(j * PCHUNK, PCHUNK)], prb)
        pltpu.sync_copy(ps_hbm.at[pl.ds(j * PCHUNK, PCHUNK)], psb)

        def inner(i, c2):
            for u in range(4):
                o = (i * 4 + u) * L
                r = prb[pl.ds(o, L)]
                s = psb[pl.ds(o, L)]
                q = r >> 7
                k = q >> 5
                c = r & (CC - 1)
                mfull = (r >= 0) & (q < NQ) & ((q & (NW - 1)) == wid)
                plsc.store_scatter(
                    buf3,
                    [jnp.where(mfull, k * STENCIL + s, 0), jnp.where(mfull, c, 0)],
                    sent,
                    mask=mfull,
                )
                mtail = (r >= 0) & (q == NQ) & (wid == TAIL_OWNER)
                plsc.store_scatter(
                    buft,
                    [s, jnp.where(mtail, c, 0)],
                    sent,
                    mask=mtail,
                )
            return c2

        return lax.fori_loop(0, PCHUNK // L // 4, inner, carry)

    lax.fori_loop(0, n_pchunks, pchunk, 0)

    def wrblk(k, carry):
        q = wid + NW * k

        @pl.when(q < NQ)
        def _():
            pltpu.sync_copy(buf3.at[pl.ds(k * STENCIL, STENCIL)], idx2_hbm.at[q])

        @pl.when(q == NQ)
        def _():
            pltpu.sync_copy(buft, idx2_hbm.at[NQ])

        return carry

    lax.fori_loop(0, KB, wrblk, 0)


def _gather_chunk(table, ibuf, obuf, width):
    for s in range(STENCIL):
        for w in range(width // L):
            v = ibuf[s, pl.ds(w * L, L)]
            obuf[s, pl.ds(w * L, L)] = plsc.load_gather(table, [v])


def _gather_body(
    inp_hbm,
    idx2_hbm,
    out_hbm,
    table,
    ib0,
    ib1,
    ibt,
    ob0,
    ob1,
    obt,
    is0,
    is1,
    ist,
    os0,
    os1,
    ost,
):
    wid = lax.axis_index("s") * NC + lax.axis_index("c")
    b = wid // 2
    h = wid % 2

    def start_in(q, ib, sem):
        pltpu.async_copy(idx2_hbm.at[q], ib, sem)

    def wait_in(q, ib, sem):
        pltpu.make_async_copy(idx2_hbm.at[q], ib, sem).wait()

    def start_out(q, ob, sem):
        pltpu.async_copy(ob, out_hbm.at[b, :, pl.ds(q * CC, CC)], sem)

    def wait_out(q, ob, sem):
        pltpu.make_async_copy(ob, out_hbm.at[b, :, pl.ds(q * CC, CC)], sem).wait()

    start_in(h, ib0, is0)
    start_in(h + 2, ib1, is1)
    pltpu.sync_copy(
        inp_hbm.at[pl.ds(b * N_CELLS, N_CELLS)], table.at[pl.ds(0, N_CELLS)]
    )
    table[pl.ds(N_CELLS, L)] = jnp.zeros((L,), jnp.float32)

    @pl.when(h == 1)
    def _():
        pltpu.sync_copy(idx2_hbm.at[NQ], ibt)
        _gather_chunk(table, ibt, obt, TAILC)
        pltpu.async_copy(obt, out_hbm.at[b, :, pl.ds(NQ * CC, TAILC)], ost)

    def pair(j, carry):
        q0 = 4 * j + h
        wait_in(q0, ib0, is0)

        @pl.when(j > 0)
        def _():
            wait_out(q0 - 4, ob0, os0)

        _gather_chunk(table, ib0, ob0, CC)
        start_out(q0, ob0, os0)

        @pl.when(q0 + 4 < NQ)
        def _():
            start_in(q0 + 4, ib0, is0)

        q1 = q0 + 2
        wait_in(q1, ib1, is1)

        @pl.when(j > 0)
        def _():
            wait_out(q1 - 4, ob1, os1)

        _gather_chunk(table, ib1, ob1, CC)
        start_out(q1, ob1, os1)

        @pl.when(q1 + 4 < NQ)
        def _():
            start_in(q1 + 4, ib1, is1)

        return carry

    lax.fori_loop(0, NPAIR, pair, 0)

    @pl.when(h == 0)
    def _():
        wait_in(NQ - 1, ib0, is0)
        wait_out(NQ - 5, ob0, os0)
        _gather_chunk(table, ib0, ob0, CC)
        start_out(NQ - 1, ob0, os0)

    wait_out(jnp.where(h == 0, NQ - 1, NQ - 4), ob0, os0)
    wait_out(NQ - 3 + h, ob1, os1)

    @pl.when(h == 1)
    def _():
        pltpu.make_async_copy(
            obt, out_hbm.at[b, :, pl.ds(NQ * CC, TAILC)], ost
        ).wait()


def kernel(inputs, indexes, positions):
    idxT = indexes.T
    pr = positions[:, 0].astype(jnp.int32)
    ps = positions[:, 1].astype(jnp.int32)
    p = pr.shape[0]
    n_pchunks = max(1, (p + PCHUNK - 1) // PCHUNK)
    pad = jnp.full((n_pchunks * PCHUNK - p,), -1, jnp.int32)
    pr_pad = jnp.concatenate([pr, pad])
    ps_pad = jnp.concatenate([ps, pad])
    inp = inputs.reshape(-1)

    idx2 = pl.kernel(
        functools.partial(_remap_body, n_pchunks),
        out_type=jax.ShapeDtypeStruct((NQ + 1, STENCIL, CC), jnp.int32),
        mesh=_mesh(),
        scratch_types=[
            pltpu.VMEM((KB * STENCIL, CC), jnp.int32),
            pltpu.VMEM((STENCIL, CC), jnp.int32),
            pltpu.VMEM((STENCIL, TAILC), jnp.int32),
            pltpu.VMEM((PCHUNK,), jnp.int32),
            pltpu.VMEM((PCHUNK,), jnp.int32),
        ],
        compiler_params=pltpu.CompilerParams(needs_layout_passes=False),
    )(idxT, pr_pad, ps_pad)

    out = pl.kernel(
        _gather_body,
        out_type=jax.ShapeDtypeStruct((BATCH, STENCIL, N_CELLS), jnp.float32),
        mesh=_mesh(),
        scratch_types=[
            pltpu.VMEM((TBL_W,), jnp.float32),
            pltpu.VMEM((STENCIL, CC), jnp.int32),
            pltpu.VMEM((STENCIL, CC), jnp.int32),
            pltpu.VMEM((STENCIL, CC), jnp.int32),
            pltpu.VMEM((STENCIL, CC), jnp.float32),
            pltpu.VMEM((STENCIL, CC), jnp.float32),
            pltpu.VMEM((STENCIL, TAILC), jnp.float32),
            pltpu.SemaphoreType.DMA,
            pltpu.SemaphoreType.DMA,
            pltpu.SemaphoreType.DMA,
            pltpu.SemaphoreType.DMA,
            pltpu.SemaphoreType.DMA,
            pltpu.SemaphoreType.DMA,
        ],
        compiler_params=pltpu.CompilerParams(
            needs_layout_passes=False, use_tc_tiling_on_sc=True
        ),
    )(inp, idx2)

    return out.transpose(0, 2, 1)


# speedup vs baseline: 42.5723x; 1.0147x over previous
import functools

import jax
import jax.numpy as jnp
from jax import lax
from jax.experimental import pallas as pl
from jax.experimental.pallas import tpu as pltpu
from jax.experimental.pallas import tpu_sc as plsc

N_CELLS = 100000
STENCIL = 25
BATCH = 16

NC = 2
NS = 16
L = 16
NW = NC * NS

SENTINEL = N_CELLS
TBL_W = N_CELLS + L
PCHUNK = 2048
CC = 128
CW = CC * STENCIL
NQ = N_CELLS // CC
TAILC = N_CELLS - NQ * CC
KB = 25
TAIL_OWNER = NQ - (NQ // NW) * NW
NPAIR = 195


def _mesh():
    return plsc.VectorSubcoreMesh(
        core_axis_name="c", subcore_axis_name="s", num_cores=NC, num_subcores=NS
    )


def _remap_body(
    n_pchunks, idxT_hbm, pr_hbm, ps_hbm, idx2_hbm, buf3, buft, buft32, prb, psb, bsem
):
    wid = lax.axis_index("s") * NC + lax.axis_index("c")
    sent = jnp.full((L,), SENTINEL, jnp.int32)

    def ldblk(k, carry):
        q = wid + NW * k

        @pl.when(q < NQ)
        def _():
            pltpu.async_copy(
                idxT_hbm.at[:, pl.ds(q * CC, CC)],
                buf3.at[pl.ds(k * STENCIL, STENCIL)],
                bsem,
            )

        @pl.when(q == NQ)
        def _():
            pltpu.sync_copy(idxT_hbm.at[:, pl.ds(NQ * CC, TAILC)], buft32)
            for s in range(STENCIL):
                for w in range(CC // L):
                    if w < TAILC // L:
                        buft[s, pl.ds(w * L, L)] = buft32[s, pl.ds(w * L, L)]
                    else:
                        buft[s, pl.ds(w * L, L)] = sent

        return carry

    lax.fori_loop(0, KB, ldblk, 0)

    def lddrain(k, carry):
        q = wid + NW * k

        @pl.when(q < NQ)
        def _():
            pltpu.make_async_copy(
                idxT_hbm.at[:, pl.ds(q * CC, CC)],
                buf3.at[pl.ds(k * STENCIL, STENCIL)],
                bsem,
            ).wait()

        return carry

    lax.fori_loop(0, KB, lddrain, 0)

    def pchunk(j, carry):
        pltpu.sync_copy(pr_hbm.at[pl.ds(j * PCHUNK, PCHUNK)], prb)
        pltpu.sync_copy(ps_hbm.at[pl.ds(j * PCHUNK, PCHUNK)], psb)

        def inner(i, c2):
            for u in range(4):
                o = (i * 4 + u) * L
                r = prb[pl.ds(o, L)]
                s = psb[pl.ds(o, L)]
                q = r >> 7
                k = q >> 5
                c = r & (CC - 1)
                mfull = (r >= 0) & (q < NQ) & ((q & (NW - 1)) == wid)
                plsc.store_scatter(
                    buf3,
                    [jnp.where(mfull, k * STENCIL + s, 0), jnp.where(mfull, c, 0)],
                    sent,
                    mask=mfull,
                )
                mtail = (r >= 0) & (q == NQ) & (wid == TAIL_OWNER)
                plsc.store_scatter(
                    buft,
                    [s, jnp.where(mtail, c, 0)],
                    sent,
                    mask=mtail,
                )
            return c2

        return lax.fori_loop(0, PCHUNK // L // 4, inner, carry)

    lax.fori_loop(0, n_pchunks, pchunk, 0)

    def wrblk(k, carry):
        q = wid + NW * k

        @pl.when(q < NQ)
        def _():
            pltpu.async_copy(
                buf3.at[pl.ds(k * STENCIL, STENCIL)], idx2_hbm.at[q], bsem
            )

        @pl.when(q == NQ)
        def _():
            pltpu.sync_copy(buft, idx2_hbm.at[NQ])

        return carry

    lax.fori_loop(0, KB, wrblk, 0)

    def wrdrain(k, carry):
        q = wid + NW * k

        @pl.when(q < NQ)
        def _():
            pltpu.make_async_copy(
                buf3.at[pl.ds(k * STENCIL, STENCIL)], idx2_hbm.at[q], bsem
            ).wait()

        return carry

    lax.fori_loop(0, KB, wrdrain, 0)


def _gather_chunk(table, ibuf, obuf, width):
    for s in range(STENCIL):
        for w in range(width // L):
            v = ibuf[s, pl.ds(w * L, L)]
            obuf[s, pl.ds(w * L, L)] = plsc.load_gather(table, [v])


def _gather_body(
    inp_hbm,
    idx2_hbm,
    out_hbm,
    table,
    ib0,
    ib1,
    ibt,
    ob0,
    ob1,
    obt,
    is0,
    is1,
    ist,
    os0,
    os1,
    ost,
):
    wid = lax.axis_index("s") * NC + lax.axis_index("c")
    b = wid // 2
    h = wid % 2

    def start_in(q, ib, sem):
        pltpu.async_copy(idx2_hbm.at[q], ib, sem)

    def wait_in(q, ib, sem):
        pltpu.make_async_copy(idx2_hbm.at[q], ib, sem).wait()

    def start_out(q, ob, sem):
        pltpu.async_copy(ob, out_hbm.at[b, :, pl.ds(q * CC, CC)], sem)

    def wait_out(q, ob, sem):
        pltpu.make_async_copy(ob, out_hbm.at[b, :, pl.ds(q * CC, CC)], sem).wait()

    start_in(h, ib0, is0)
    start_in(h + 2, ib1, is1)
    pltpu.sync_copy(
        inp_hbm.at[pl.ds(b * N_CELLS, N_CELLS)], table.at[pl.ds(0, N_CELLS)]
    )
    table[pl.ds(N_CELLS, L)] = jnp.zeros((L,), jnp.float32)

    @pl.when(h == 1)
    def _():
        pltpu.sync_copy(idx2_hbm.at[NQ], ibt)
        _gather_chunk(table, ibt, obt, TAILC)
        pltpu.async_copy(obt, out_hbm.at[b, :, pl.ds(NQ * CC, TAILC)], ost)

    def pair(j, carry):
        q0 = 4 * j + h
        wait_in(q0, ib0, is0)

        @pl.when(j > 0)
        def _():
            wait_out(q0 - 4, ob0, os0)

        _gather_chunk(table, ib0, ob0, CC)
        start_out(q0, ob0, os0)

        @pl.when(q0 + 4 < NQ)
        def _():
            start_in(q0 + 4, ib0, is0)

        q1 = q0 + 2
        wait_in(q1, ib1, is1)

        @pl.when(j > 0)
        def _():
            wait_out(q1 - 4, ob1, os1)

        _gather_chunk(table, ib1, ob1, CC)
        start_out(q1, ob1, os1)

        @pl.when(q1 + 4 < NQ)
        def _():
            start_in(q1 + 4, ib1, is1)

        return carry

    lax.fori_loop(0, NPAIR, pair, 0)

    @pl.when(h == 0)
    def _():
        wait_in(NQ - 1, ib0, is0)
        wait_out(NQ - 5, ob0, os0)
        _gather_chunk(table, ib0, ob0, CC)
        start_out(NQ - 1, ob0, os0)

    wait_out(jnp.where(h == 0, NQ - 1, NQ - 4), ob0, os0)
    wait_out(NQ - 3 + h, ob1, os1)

    @pl.when(h == 1)
    def _():
        pltpu.make_async_copy(
            obt, out_hbm.at[b, :, pl.ds(NQ * CC, TAILC)], ost
        ).wait()


def kernel(inputs, indexes, positions):
    idxT = indexes.T
    pr = positions[:, 0].astype(jnp.int32)
    ps = positions[:, 1].astype(jnp.int32)
    p = pr.shape[0]
    n_pchunks = max(1, (p + PCHUNK - 1) // PCHUNK)
    pad = jnp.full((n_pchunks * PCHUNK - p,), -1, jnp.int32)
    pr_pad = jnp.concatenate([pr, pad])
    ps_pad = jnp.concatenate([ps, pad])
    inp = inputs.reshape(-1)

    idx2 = pl.kernel(
        functools.partial(_remap_body, n_pchunks),
        out_type=jax.ShapeDtypeStruct((NQ + 1, STENCIL, CC), jnp.int32),
        mesh=_mesh(),
        scratch_types=[
            pltpu.VMEM((KB * STENCIL, CC), jnp.int32),
            pltpu.VMEM((STENCIL, CC), jnp.int32),
            pltpu.VMEM((STENCIL, TAILC), jnp.int32),
            pltpu.VMEM((PCHUNK,), jnp.int32),
            pltpu.VMEM((PCHUNK,), jnp.int32),
            pltpu.SemaphoreType.DMA,
        ],
        compiler_params=pltpu.CompilerParams(needs_layout_passes=False),
    )(idxT, pr_pad, ps_pad)

    out = pl.kernel(
        _gather_body,
        out_type=jax.ShapeDtypeStruct((BATCH, STENCIL, N_CELLS), jnp.float32),
        mesh=_mesh(),
        scratch_types=[
            pltpu.VMEM((TBL_W,), jnp.float32),
            pltpu.VMEM((STENCIL, CC), jnp.int32),
            pltpu.VMEM((STENCIL, CC), jnp.int32),
            pltpu.VMEM((STENCIL, CC), jnp.int32),
            pltpu.VMEM((STENCIL, CC), jnp.float32),
            pltpu.VMEM((STENCIL, CC), jnp.float32),
            pltpu.VMEM((STENCIL, TAILC), jnp.float32),
            pltpu.SemaphoreType.DMA,
            pltpu.SemaphoreType.DMA,
            pltpu.SemaphoreType.DMA,
            pltpu.SemaphoreType.DMA,
            pltpu.SemaphoreType.DMA,
            pltpu.SemaphoreType.DMA,
        ],
        compiler_params=pltpu.CompilerParams(
            needs_layout_passes=False, use_tc_tiling_on_sc=True
        ),
    )(inp, idx2)

    return out.transpose(0, 2, 1)
